# baseline (speedup 1.0000x reference)
import functools
import numpy as np
import jax
import jax.numpy as jnp
from jax import lax
from jax.experimental import pallas as pl
from jax.experimental.pallas import tpu as pltpu
from jax.experimental.pallas import tpu_sc as plsc

D = 768
EPS = 1e-5
NSL = D // 16
NW = 32
CH = 16


def _make_pe(n):
    position = np.arange(n, dtype=np.float32)[:, None]
    div_term = np.exp(
        np.arange(0, D, 2, dtype=np.float32) * -(np.log(10000.0) / D)
    )
    pe = np.zeros((n, D), dtype=np.float32)
    pe[:, 0::2] = np.sin(position * div_term)
    pe[:, 1::2] = np.cos(position * div_term)
    return jnp.asarray(pe)


def _sum_bcast16(tmp_ref, vec):
    it = lax.broadcasted_iota(jnp.int32, (16,), 0)
    for shift in (8, 4, 2, 1):
        tmp_ref[...] = vec
        vec = vec + plsc.load_gather(tmp_ref, [it ^ shift])
    return vec


def _rsqrt16(v):
    iv = plsc.bitcast(v, jnp.int32)
    r = plsc.bitcast(jnp.int32(0x5F3759DF) - (iv >> 1), jnp.float32)
    for _ in range(3):
        r = r * (1.5 - 0.5 * v * r * r)
    return r


def _sc_header_body(T, P, NBP, TPW,
                    idx_hbm, tab_hbm, g_hbm, b_hbm, pe_hbm, pk_hbm,
                    out_hbm,
                    idx_v, pe_v, pk_v, tpkb_v, g_v, b_v, rows_v, out_v,
                    tmp_v, sem):
    c_i = lax.axis_index("c")
    s_i = lax.axis_index("s")
    wid = s_i * 2 + c_i

    pltpu.sync_copy(idx_hbm.at[pl.ds(wid * TPW * NBP, TPW * NBP)], idx_v)
    pltpu.sync_copy(pe_hbm.at[pl.ds(wid * TPW, TPW)], pe_v)
    pltpu.sync_copy(pk_hbm, pk_v)
    pltpu.sync_copy(g_hbm, g_v)
    pltpu.sync_copy(b_hbm, b_v)

    for p in range(P):
        for j in range(NSL):
            x = pk_v[p, pl.ds(j * 16, 16)]
            t = 1.0 - 2.0 / (jnp.exp(2.0 * x) + 1.0)
            tpkb_v[p, pl.ds(j * 16, 16)] = t + b_v[pl.ds(j * 16, 16)]

    lane = lax.broadcasted_iota(jnp.int32, (16,), 0)
    for ti in range(TPW):
        t_glob = wid * TPW + ti

        def chunk_body(ci, _):
            idx16 = idx_v[pl.ds(ti * NBP + ci * CH, CH)]
            pltpu.async_copy(tab_hbm.at[idx16], rows_v, sem).wait()

            def row_body(i, _):
                def halves(j):
                    iv = rows_v[i, pl.ds(j * 16, 16)]
                    lo = plsc.bitcast(iv << 16, jnp.float32)
                    hi = plsc.bitcast(iv & jnp.int32(-65536), jnp.float32)
                    return lo, hi

                lo0, hi0 = halves(0)
                sv = lo0 + hi0
                qv = lo0 * lo0 + hi0 * hi0
                for j in range(1, NSL // 2):
                    lo, hi = halves(j)
                    sv = sv + lo + hi
                    qv = qv + lo * lo + hi * hi
                m16 = _sum_bcast16(tmp_v, sv) * np.float32(1.0 / D)
                q16 = _sum_bcast16(tmp_v, qv) * np.float32(1.0 / D)
                r16 = _rsqrt16(q16 - m16 * m16 + np.float32(EPS))

                p_idx = lax.rem(ci * CH + i, P)
                i16 = jnp.full((16,), i, jnp.int32)
                for j in range(NSL // 2):
                    lo, hi = halves(j)
                    sle = pl.ds(j * 16, 16)
                    slo = pl.ds(D // 2 + j * 16, 16)
                    oe = ((lo - m16) * r16 * g_v[sle]
                          + pe_v[ti, sle] + tpkb_v[p_idx, sle])
                    oo = ((hi - m16) * r16 * g_v[slo]
                          + pe_v[ti, slo] + tpkb_v[p_idx, slo])
                    pos = j * 32 + 2 * lane
                    plsc.store_scatter(out_v, [i16, pos], oe)
                    plsc.store_scatter(out_v, [i16, pos + 1], oo)
                return 0

            lax.fori_loop(0, CH, row_body, 0)
            oidx16 = (ci * CH + lane) * T + t_glob
            pltpu.async_copy(out_v, out_hbm.at[oidx16], sem).wait()
            return 0

        lax.fori_loop(0, NBP // CH, chunk_body, 0)


def _sc_header(idx_t_flat, header_table, hg, hb, pe_h, pk, nbp):
    T = pe_h.shape[0]
    P = pk.shape[0]
    tpw = T // NW
    body = functools.partial(_sc_header_body, T, P, nbp, tpw)
    return pl.kernel(
        body,
        out_type=jax.ShapeDtypeStruct((nbp * T, D), jnp.float32),
        mesh=plsc.VectorSubcoreMesh(core_axis_name="c", subcore_axis_name="s"),
        scratch_types=[
            pltpu.VMEM((tpw * nbp,), jnp.int32),
            pltpu.VMEM((tpw, D), jnp.float32),
            pltpu.VMEM((P, D), jnp.float32),
            pltpu.VMEM((P, D), jnp.float32),
            pltpu.VMEM((D,), jnp.float32),
            pltpu.VMEM((D,), jnp.float32),
            pltpu.VMEM((CH, D // 2), jnp.int32),
            pltpu.VMEM((CH, D), jnp.float32),
            pltpu.VMEM((16,), jnp.float32),
            pltpu.SemaphoreType.DMA,
        ],
        compiler_params=pltpu.CompilerParams(needs_layout_passes=False),
    )(idx_t_flat, header_table, hg, hb, pe_h, pk)


def _tc_payload_body(x_ref, g4_ref, w5_ref, pe_p_ref, pk_ref, p_out_ref):
    pk = jnp.tanh(pk_ref[0])
    x = x_ref[0, 0]
    xm = jnp.roll(x, 1, axis=0)
    xp = jnp.roll(x, -1, axis=0)
    x3 = jnp.concatenate([xm, x, xp], axis=1)
    t4 = jnp.dot(x3, g4_ref[...], preferred_element_type=jnp.float32)
    v = jnp.sum(x3 * t4[:, :3], axis=1, keepdims=True)
    m = t4[:, 3:4]
    r = jax.lax.rsqrt(v + EPS)
    x5 = jnp.concatenate([x3 * r, -(m * r), jnp.ones_like(x)], axis=1)
    z = jnp.dot(x5, w5_ref[...], preferred_element_type=jnp.float32)
    e = jax.lax.erf(z * np.float32(1.0 / np.sqrt(2.0)))
    zz = z * (0.5 * e + 0.5)
    p_out_ref[0, 0] = zz + pe_p_ref[...] + pk


def kernel(headers, payloads, header_table, header_ln_g, header_ln_b,
           conv_w, conv_ln_g, conv_ln_b, packet_table):
    B, P, T = headers.shape
    L = payloads.shape[2]

    x = payloads.reshape(B, P, L, 1)
    pe_h = _make_pe(T)
    pe_p = _make_pe(L)

    w = conv_w[:, 0, :].T
    s = jnp.sum(w, axis=1, keepdims=True) * np.float32(1.0 / D)
    gram = (w @ w.T) * np.float32(1.0 / D)
    g4 = jnp.concatenate([gram - s @ s.T, s], axis=1)
    w5 = jnp.concatenate(
        [w * conv_ln_g[None, :], conv_ln_g[None, :], conv_ln_b[None, :]],
        axis=0,
    )

    const = lambda shape: pl.BlockSpec(shape, lambda b, p: (0,) * len(shape))

    p_out = pl.pallas_call(
        _tc_payload_body,
        grid=(B, P),
        in_specs=[
            pl.BlockSpec((1, 1, L, 1), lambda b, p: (b, p, 0, 0)),
            const((3, 4)),
            const((5, D)),
            const((L, D)),
            pl.BlockSpec((1, 1, D), lambda b, p: (p, 0, 0)),
        ],
        out_specs=pl.BlockSpec((1, 1, L, D), lambda b, p: (b, p, 0, 0)),
        out_shape=jax.ShapeDtypeStruct((B, P, L, D), jnp.float32),
    )(x, g4, w5, pe_p, packet_table.reshape(P, 1, D))

    idx_t = headers.astype(jnp.int32).reshape(B * P, T).T.reshape(-1)
    split = lambda a: jnp.concatenate([a[..., 0::2], a[..., 1::2]], axis=-1)
    tab_i32 = jax.lax.bitcast_convert_type(
        header_table.astype(jnp.bfloat16).reshape(256, D // 2, 2), jnp.int32
    )
    h_flat = _sc_header(
        idx_t, tab_i32,
        split(header_ln_g), split(header_ln_b), split(pe_h),
        split(packet_table), B * P,
    )
    return h_flat.reshape(B, P, T, D), p_out


# speedup vs baseline: 1.0594x; 1.0594x over previous
import functools
import numpy as np
import jax
import jax.numpy as jnp
from jax import lax
from jax.experimental import pallas as pl
from jax.experimental.pallas import tpu as pltpu
from jax.experimental.pallas import tpu_sc as plsc

D = 768
EPS = 1e-5
NSL = D // 16
NW = 32
CH = 16


def _make_pe(n):
    position = np.arange(n, dtype=np.float32)[:, None]
    div_term = np.exp(
        np.arange(0, D, 2, dtype=np.float32) * -(np.log(10000.0) / D)
    )
    pe = np.zeros((n, D), dtype=np.float32)
    pe[:, 0::2] = np.sin(position * div_term)
    pe[:, 1::2] = np.cos(position * div_term)
    return jnp.asarray(pe)


def _sum_bcast16(tmp_ref, vec):
    it = lax.broadcasted_iota(jnp.int32, (16,), 0)
    for shift in (8, 4, 2, 1):
        tmp_ref[...] = vec
        vec = vec + plsc.load_gather(tmp_ref, [it ^ shift])
    return vec


def _rsqrt16(v):
    iv = plsc.bitcast(v, jnp.int32)
    r = plsc.bitcast(jnp.int32(0x5F3759DF) - (iv >> 1), jnp.float32)
    for _ in range(3):
        r = r * (1.5 - 0.5 * v * r * r)
    return r


def _sc_header_body(T, P, NBP, TPW,
                    idx_hbm, tab_hbm, g_hbm, b_hbm, pe_hbm, pk_hbm,
                    out_hbm,
                    idx_v, pe_v, pk_v, tpkb_v, g_v, b_v, rows_v, tmp_v, sem):
    c_i = lax.axis_index("c")
    s_i = lax.axis_index("s")
    wid = s_i * 2 + c_i

    pltpu.sync_copy(idx_hbm.at[pl.ds(wid * TPW * NBP, TPW * NBP)], idx_v)
    pltpu.sync_copy(pe_hbm.at[pl.ds(wid * TPW, TPW)], pe_v)
    pltpu.sync_copy(pk_hbm, pk_v)
    pltpu.sync_copy(g_hbm, g_v)
    pltpu.sync_copy(b_hbm, b_v)

    for p in range(P):
        for j in range(NSL):
            x = pk_v[p, pl.ds(j * 16, 16)]
            t = 1.0 - 2.0 / (jnp.exp(2.0 * x) + 1.0)
            tpkb_v[p, pl.ds(j * 16, 16)] = t + b_v[pl.ds(j * 16, 16)]

    lane = lax.broadcasted_iota(jnp.int32, (16,), 0)
    for ti in range(TPW):
        t_glob = wid * TPW + ti

        def chunk_body(ci, _):
            idx16 = idx_v[pl.ds(ti * NBP + ci * CH, CH)]
            pltpu.async_copy(tab_hbm.at[idx16], rows_v, sem).wait()

            def row_body(i, _):
                h0 = rows_v[i, pl.ds(0, 16)]
                sv = h0
                qv = h0 * h0
                for j in range(1, NSL):
                    h = rows_v[i, pl.ds(j * 16, 16)]
                    sv = sv + h
                    qv = qv + h * h
                m16 = _sum_bcast16(tmp_v, sv) * np.float32(1.0 / D)
                q16 = _sum_bcast16(tmp_v, qv) * np.float32(1.0 / D)
                r16 = _rsqrt16(q16 - m16 * m16 + np.float32(EPS))

                p_idx = lax.rem(ci * CH + i, P)
                for j in range(NSL):
                    sl = pl.ds(j * 16, 16)
                    rows_v[i, sl] = ((rows_v[i, sl] - m16) * r16 * g_v[sl]
                                     + pe_v[ti, sl] + tpkb_v[p_idx, sl])
                return 0

            lax.fori_loop(0, CH, row_body, 0)
            oidx16 = (ci * CH + lane) * T + t_glob
            pltpu.async_copy(rows_v, out_hbm.at[oidx16], sem).wait()
            return 0

        lax.fori_loop(0, NBP // CH, chunk_body, 0)


def _sc_header(idx_t_flat, header_table, hg, hb, pe_h, pk, nbp):
    T = pe_h.shape[0]
    P = pk.shape[0]
    tpw = T // NW
    body = functools.partial(_sc_header_body, T, P, nbp, tpw)
    return pl.kernel(
        body,
        out_type=jax.ShapeDtypeStruct((nbp * T, D), jnp.float32),
        mesh=plsc.VectorSubcoreMesh(core_axis_name="c", subcore_axis_name="s"),
        scratch_types=[
            pltpu.VMEM((tpw * nbp,), jnp.int32),
            pltpu.VMEM((tpw, D), jnp.float32),
            pltpu.VMEM((P, D), jnp.float32),
            pltpu.VMEM((P, D), jnp.float32),
            pltpu.VMEM((D,), jnp.float32),
            pltpu.VMEM((D,), jnp.float32),
            pltpu.VMEM((CH, D), jnp.float32),
            pltpu.VMEM((16,), jnp.float32),
            pltpu.SemaphoreType.DMA,
        ],
        compiler_params=pltpu.CompilerParams(needs_layout_passes=False),
    )(idx_t_flat, header_table, hg, hb, pe_h, pk)


def _tc_payload_body(x_ref, g4_ref, w5_ref, pe_p_ref, pk_ref, p_out_ref):
    pk = jnp.tanh(pk_ref[0])
    x = x_ref[0, 0]
    xm = jnp.roll(x, 1, axis=0)
    xp = jnp.roll(x, -1, axis=0)
    x3 = jnp.concatenate([xm, x, xp], axis=1)
    t4 = jnp.dot(x3, g4_ref[...], preferred_element_type=jnp.float32)
    v = jnp.sum(x3 * t4[:, :3], axis=1, keepdims=True)
    m = t4[:, 3:4]
    r = jax.lax.rsqrt(v + EPS)
    x5 = jnp.concatenate([x3 * r, -(m * r), jnp.ones_like(x)], axis=1)
    z = jnp.dot(x5, w5_ref[...], preferred_element_type=jnp.float32)
    e = jax.lax.erf(z * np.float32(1.0 / np.sqrt(2.0)))
    zz = z * (0.5 * e + 0.5)
    p_out_ref[0, 0] = zz + pe_p_ref[...] + pk


def kernel(headers, payloads, header_table, header_ln_g, header_ln_b,
           conv_w, conv_ln_g, conv_ln_b, packet_table):
    B, P, T = headers.shape
    L = payloads.shape[2]

    x = payloads.reshape(B, P, L, 1)
    pe_h = _make_pe(T)
    pe_p = _make_pe(L)

    w = conv_w[:, 0, :].T
    s = jnp.sum(w, axis=1, keepdims=True) * np.float32(1.0 / D)
    gram = (w @ w.T) * np.float32(1.0 / D)
    g4 = jnp.concatenate([gram - s @ s.T, s], axis=1)
    w5 = jnp.concatenate(
        [w * conv_ln_g[None, :], conv_ln_g[None, :], conv_ln_b[None, :]],
        axis=0,
    )

    const = lambda shape: pl.BlockSpec(shape, lambda b, p: (0,) * len(shape))

    p_out = pl.pallas_call(
        _tc_payload_body,
        grid=(B, P),
        in_specs=[
            pl.BlockSpec((1, 1, L, 1), lambda b, p: (b, p, 0, 0)),
            const((3, 4)),
            const((5, D)),
            const((L, D)),
            pl.BlockSpec((1, 1, D), lambda b, p: (p, 0, 0)),
        ],
        out_specs=pl.BlockSpec((1, 1, L, D), lambda b, p: (b, p, 0, 0)),
        out_shape=jax.ShapeDtypeStruct((B, P, L, D), jnp.float32),
    )(x, g4, w5, pe_p, packet_table.reshape(P, 1, D))

    idx_t = headers.astype(jnp.int32).reshape(B * P, T).T.reshape(-1)
    h_flat = _sc_header(
        idx_t, header_table, header_ln_g, header_ln_b, pe_h, packet_table,
        B * P,
    )
    return h_flat.reshape(B, P, T, D), p_out
